# baseline (speedup 1.0000x reference)
import functools

import jax
import jax.numpy as jnp
from jax import lax
from jax.experimental import pallas as pl
from jax.experimental.pallas import tpu as pltpu
from jax.experimental.pallas import tpu_sc as plsc

B, T, S = 16, 2048, 8
D = 64
N = B * T
NC, NS, L = 2, 16, 16
NW = NC * NS
TPW = N // NW
C = 64
NCHUNK = TPW // C


def _sac_body(idx_hbm, amap_flat_hbm, table_hbm, pos_hbm,
              out_hbm, syn_hbm,
              idx_v, offs_v, syn_v, rows_v, pos_v, out_v,
              sem_syn_in, sem_syn_out, sem_rows, sem_pos, sem_out):
    wid = lax.axis_index("s") * NC + lax.axis_index("c")
    base = wid * TPW
    pos_base = (wid % 2) * TPW

    pltpu.sync_copy(idx_hbm.at[pl.ds(base, TPW)], idx_v)

    lane = lax.iota(jnp.int32, L)
    pat0 = lax.shift_right_logical(lane, 3)
    jcol = lax.bitwise_and(lane, S - 1)

    def do_expand(g, _):
        iv = idx_v[pl.ds(g * L, L)]
        for h in range(L // 2):
            tok = jnp.take_along_axis(iv, pat0 + 2 * h, axis=0,
                                      mode="promise_in_bounds")
            offs_v[pl.ds((g * L + 2 * h) * S, L)] = tok * S + jcol
        return 0

    lax.fori_loop(0, TPW // L, do_expand, 0)

    pltpu.async_copy(amap_flat_hbm.at[offs_v], syn_v, sem_syn_in).wait()
    syn_out = pltpu.make_async_copy(
        syn_v, syn_hbm.at[pl.ds(base * S, TPW * S)], sem_syn_out)
    syn_out.start()

    def do_chunk(c, _):
        pltpu.async_copy(table_hbm.at[syn_v.at[pl.ds(c * C * S, C * S)]],
                         rows_v, sem_rows).wait()
        pltpu.sync_copy(pos_hbm.at[pl.ds(pos_base + c * C, C)], pos_v)

        def do_token(t, _):
            t8 = t * S
            for k in range(D // L):
                sl = pl.ds(k * L, L)
                a0 = rows_v[t8 + 0, sl] + rows_v[t8 + 1, sl]
                a1 = rows_v[t8 + 2, sl] + rows_v[t8 + 3, sl]
                a2 = rows_v[t8 + 4, sl] + rows_v[t8 + 5, sl]
                a3 = rows_v[t8 + 6, sl] + rows_v[t8 + 7, sl]
                acc = (a0 + a1) + (a2 + a3)
                out_v[t, sl] = acc * 0.125 + pos_v[t, sl]
            return 0

        lax.fori_loop(0, C, do_token, 0)
        pltpu.sync_copy(out_v, out_hbm.at[pl.ds(base + c * C, C)])
        return 0

    lax.fori_loop(0, NCHUNK, do_chunk, 0)
    syn_out.wait()


@jax.jit
def _sac(idx_flat, amap_flat, syn_table, pos_table):
    mesh = plsc.VectorSubcoreMesh(core_axis_name="c", subcore_axis_name="s")
    return pl.kernel(
        _sac_body,
        out_type=(jax.ShapeDtypeStruct((N, D), jnp.float32),
                  jax.ShapeDtypeStruct((N * S,), jnp.int32)),
        mesh=mesh,
        compiler_params=pltpu.CompilerParams(use_tc_tiling_on_sc=False),
        scratch_types=[
            pltpu.VMEM((TPW,), jnp.int32),
            pltpu.VMEM((TPW * S,), jnp.int32),
            pltpu.VMEM((TPW * S,), jnp.int32),
            pltpu.VMEM((C * S, D), jnp.float32),
            pltpu.VMEM((C, D), jnp.float32),
            pltpu.VMEM((C, D), jnp.float32),
            pltpu.SemaphoreType.DMA,
            pltpu.SemaphoreType.DMA,
            pltpu.SemaphoreType.DMA,
            pltpu.SemaphoreType.DMA,
            pltpu.SemaphoreType.DMA,
        ],
    )(idx_flat, amap_flat, syn_table, pos_table)


def kernel(idx, address_map, syn_table, pos_table):
    out_flat, syn_flat = _sac(idx.reshape(-1), address_map.reshape(-1),
                              syn_table, pos_table)
    return out_flat.reshape(B, T, D), syn_flat.reshape(B, T, S)
